# baseline (speedup 1.0000x reference)
import functools

import jax
import jax.numpy as jnp
from jax import lax
from jax.experimental import pallas as pl
from jax.experimental.pallas import tpu as pltpu
from jax.experimental.pallas import tpu_sc as plsc

N_NODES = 65536
NH = 9
D_FEAT = 128

NC = 2
NS = 16
NW = NC * NS

TOTAL = N_NODES * NH
B_PER_W = TOTAL // NW
CHUNK = 128
NCHUNK = B_PER_W // CHUNK
NBUF = 6
KLAG = 3


def _gather_kernel(table_hbm, idx_hbm, out_hbm, idx_v, bufs, gsems, ssems):
    wid = lax.axis_index("s") * NC + lax.axis_index("c")

    pltpu.sync_copy(idx_hbm.at[wid], idx_v)

    def start_gather(j, b):
        return pltpu.async_copy(table_hbm.at[idx_v.at[j]], bufs.at[b], gsems.at[b])

    def start_scatter(j, b):
        return pltpu.async_copy(bufs.at[b], out_hbm.at[wid, j], ssems.at[b])

    for b in range(NBUF):
        start_gather(b, b)

    def wait_gather(j, b):
        pltpu.make_async_copy(table_hbm.at[idx_v.at[j]], bufs.at[b],
                              gsems.at[b]).wait()

    def wait_scatter(j, b):
        pltpu.make_async_copy(bufs.at[b], out_hbm.at[wid, j],
                              ssems.at[b]).wait()

    def group(g, carry):
        j0 = g * NBUF
        for b in range(NBUF):
            j = j0 + b
            wait_gather(j, b)
            start_scatter(j, b)
            jw = j - KLAG
            bw = (b - KLAG) % NBUF
            nj = jw + NBUF

            @pl.when(jnp.logical_and(jw >= 0, nj < NCHUNK))
            def _():
                wait_scatter(jw, bw)
                start_gather(nj, bw)

        return carry

    lax.fori_loop(0, NCHUNK // NBUF, group, 0)

    for b in range(NBUF):
        wait_scatter(NCHUNK - NBUF + b, b)


@jax.jit
def _run(table, idx):
    mesh = plsc.VectorSubcoreMesh(core_axis_name="c", subcore_axis_name="s",
                                  num_cores=NC, num_subcores=NS)
    kern = pl.kernel(
        _gather_kernel,
        out_type=jax.ShapeDtypeStruct((NW, NCHUNK, CHUNK, D_FEAT), jnp.float32),
        mesh=mesh,
        scratch_types=[
            pltpu.VMEM((NCHUNK, CHUNK), jnp.int32),
            pltpu.VMEM((NBUF, CHUNK, D_FEAT), jnp.float32),
            pltpu.SemaphoreType.DMA((NBUF,)),
            pltpu.SemaphoreType.DMA((NBUF,)),
        ],
    )
    return kern(table, idx)


def kernel(x, adjc, adjc_mask, coordinates):
    table = x.reshape(N_NODES, D_FEAT)
    idx = adjc.reshape(NW, NCHUNK, CHUNK)
    out = _run(table, idx)
    return out.reshape(1, 1, N_NODES, NH, D_FEAT)


# speedup vs baseline: 1.7878x; 1.7878x over previous
import functools

import jax
import jax.numpy as jnp
from jax import lax
from jax.experimental import pallas as pl
from jax.experimental.pallas import tpu as pltpu
from jax.experimental.pallas import tpu_sc as plsc

N_NODES = 65536
NH = 9
NH_PAD = 16
D_FEAT = 128

NC = 2
NS = 16
NW = NC * NS

TOTAL = N_NODES * NH
B_PER_W = TOTAL // NW
CHUNK = 96
NCHUNK = B_PER_W // CHUNK
NBUF = 6
KLAG = 3


def _gather_kernel(table_hbm, idx_hbm, didx_hbm, out_hbm,
                   idx_v, didx_v, bufs, gsems, ssems):
    wid = lax.axis_index("s") * NC + lax.axis_index("c")

    pltpu.sync_copy(idx_hbm.at[wid], idx_v)
    pltpu.sync_copy(didx_hbm.at[wid], didx_v)

    def gather_copy(j, b):
        return pltpu.make_async_copy(
            table_hbm.at[idx_v.at[j]], bufs.at[b], gsems.at[b])

    def scatter_copy(j, b):
        return pltpu.make_async_copy(
            bufs.at[b], out_hbm.at[didx_v.at[j]], ssems.at[b])

    for b in range(NBUF):
        gather_copy(b, b).start()

    def group(g, carry):
        j0 = g * NBUF
        for b in range(NBUF):
            j = j0 + b
            gather_copy(j, b).wait()
            scatter_copy(j, b).start()
            jw = j - KLAG
            bw = (b - KLAG) % NBUF
            nj = jw + NBUF

            @pl.when(jnp.logical_and(jw >= 0, nj < NCHUNK))
            def _():
                scatter_copy(jw, bw).wait()
                gather_copy(nj, bw).start()

        return carry

    lax.fori_loop(0, NCHUNK // NBUF, group, 0)

    for b in range(NBUF):
        scatter_copy(NCHUNK - NBUF + b, b).wait()


@jax.jit
def _run(table, idx, didx):
    mesh = plsc.VectorSubcoreMesh(core_axis_name="c", subcore_axis_name="s",
                                  num_cores=NC, num_subcores=NS)
    kern = pl.kernel(
        _gather_kernel,
        out_type=jax.ShapeDtypeStruct((N_NODES * NH_PAD, D_FEAT), jnp.float32),
        mesh=mesh,
        scratch_types=[
            pltpu.VMEM((NCHUNK, CHUNK), jnp.int32),
            pltpu.VMEM((NCHUNK, CHUNK), jnp.int32),
            pltpu.VMEM((NBUF, CHUNK, D_FEAT), jnp.float32),
            pltpu.SemaphoreType.DMA((NBUF,)),
            pltpu.SemaphoreType.DMA((NBUF,)),
        ],
    )
    return kern(table, idx, didx)


def kernel(x, adjc, adjc_mask, coordinates):
    table = x.reshape(N_NODES, D_FEAT)
    idx = adjc.reshape(NW, NCHUNK, CHUNK)
    r = lax.iota(jnp.int32, TOTAL)
    didx = ((r // NH) * NH_PAD + r % NH).reshape(NW, NCHUNK, CHUNK)
    out = _run(table, idx, didx)
    return out.reshape(N_NODES, NH_PAD, D_FEAT)[None, None, :, :NH, :]


# speedup vs baseline: 3.7794x; 2.1140x over previous
import functools

import jax
import jax.numpy as jnp
from jax import lax
from jax.experimental import pallas as pl
from jax.experimental.pallas import tpu as pltpu
from jax.experimental.pallas import tpu_sc as plsc

N_NODES = 65536
NH = 9
D_FEAT = 128

NC = 2
NS = 16
NW = NC * NS

TOTAL = N_NODES * NH
B_PER_W = TOTAL // NW
CHUNK = 128
NCHUNK = B_PER_W // CHUNK
NBUF = 6
KLAG = 3


def _gather_kernel(table_hbm, idx_hbm, out_hbm, idx_v, bufs, gsems, ssems):
    wid = lax.axis_index("s") * NC + lax.axis_index("c")

    pltpu.sync_copy(idx_hbm.at[wid], idx_v)

    def gather_copy(j, b):
        return pltpu.make_async_copy(
            table_hbm.at[idx_v.at[j]], bufs.at[b], gsems.at[b])

    def scatter_copy(j, b):
        return pltpu.make_async_copy(
            bufs.at[b], out_hbm.at[wid, j], ssems.at[b])

    for b in range(NBUF):
        gather_copy(b, b).start()

    def group(g, carry):
        j0 = g * NBUF
        for b in range(NBUF):
            j = j0 + b
            gather_copy(j, b).wait()
            scatter_copy(j, b).start()
            jw = j - KLAG
            bw = (b - KLAG) % NBUF
            nj = jw + NBUF

            @pl.when(jnp.logical_and(jw >= 0, nj < NCHUNK))
            def _():
                scatter_copy(jw, bw).wait()
                gather_copy(nj, bw).start()

        return carry

    lax.fori_loop(0, NCHUNK // NBUF, group, 0)

    for b in range(NBUF):
        scatter_copy(NCHUNK - NBUF + b, b).wait()


@jax.jit
def _run(table, idx):
    mesh = plsc.VectorSubcoreMesh(core_axis_name="c", subcore_axis_name="s",
                                  num_cores=NC, num_subcores=NS)
    kern = pl.kernel(
        _gather_kernel,
        out_type=jax.ShapeDtypeStruct((NW, NCHUNK, CHUNK, D_FEAT), jnp.float32),
        mesh=mesh,
        scratch_types=[
            pltpu.VMEM((NCHUNK, CHUNK), jnp.int32),
            pltpu.VMEM((NBUF, CHUNK, D_FEAT), jnp.float32),
            pltpu.SemaphoreType.DMA((NBUF,)),
            pltpu.SemaphoreType.DMA((NBUF,)),
        ],
    )
    return kern(table, idx)


def kernel(x, adjc, adjc_mask, coordinates):
    table = x.reshape(N_NODES, D_FEAT)
    idx = adjc.T.reshape(NW, NCHUNK, CHUNK)
    out = _run(table, idx)
    return out.reshape(NH, N_NODES, D_FEAT).transpose(1, 0, 2)[None, None]


# speedup vs baseline: 3.7840x; 1.0012x over previous
import functools

import jax
import jax.numpy as jnp
from jax import lax
from jax.experimental import pallas as pl
from jax.experimental.pallas import tpu as pltpu
from jax.experimental.pallas import tpu_sc as plsc

N_NODES = 65536
NH = 9
D_FEAT = 128

NC = 2
NS = 16
NW = NC * NS

TOTAL = N_NODES * NH
B_PER_W = TOTAL // NW
CHUNK = 128
NCHUNK = B_PER_W // CHUNK
NBUF = 6
KLAG = 3


def _gather_kernel(table_hbm, idx_hbm, out_hbm, idx_v, bufs, *sems):
    gsems = sems[:NBUF]
    ssems = sems[NBUF:]
    wid = lax.axis_index("s") * NC + lax.axis_index("c")

    pltpu.sync_copy(idx_hbm.at[wid], idx_v)

    def gather_copy(j, b):
        return pltpu.make_async_copy(
            table_hbm.at[idx_v.at[j]], bufs.at[b], gsems[b])

    def scatter_copy(j, b):
        return pltpu.make_async_copy(
            bufs.at[b], out_hbm.at[wid, j], ssems[b])

    for b in range(NBUF):
        gather_copy(b, b).start()

    def group(g, carry):
        j0 = g * NBUF
        for b in range(NBUF):
            j = j0 + b
            gather_copy(j, b).wait()
            scatter_copy(j, b).start()
            jw = j - KLAG
            bw = (b - KLAG) % NBUF
            nj = jw + NBUF

            @pl.when(jnp.logical_and(jw >= 0, nj < NCHUNK))
            def _():
                scatter_copy(jw, bw).wait()
                gather_copy(nj, bw).start()

        return carry

    lax.fori_loop(0, NCHUNK // NBUF, group, 0)

    for b in range(NBUF):
        scatter_copy(NCHUNK - NBUF + b, b).wait()


@jax.jit
def _run(table, idx):
    mesh = plsc.VectorSubcoreMesh(core_axis_name="c", subcore_axis_name="s",
                                  num_cores=NC, num_subcores=NS)
    kern = pl.kernel(
        _gather_kernel,
        out_type=jax.ShapeDtypeStruct((NW, NCHUNK, CHUNK, D_FEAT), jnp.float32),
        mesh=mesh,
        scratch_types=[
            pltpu.VMEM((NCHUNK, CHUNK), jnp.int32),
            pltpu.VMEM((NBUF, CHUNK, D_FEAT), jnp.float32),
        ] + [pltpu.SemaphoreType.DMA] * (2 * NBUF),
    )
    return kern(table, idx)


def kernel(x, adjc, adjc_mask, coordinates):
    table = x.reshape(N_NODES, D_FEAT)
    idx = adjc.T.reshape(NW, NCHUNK, CHUNK)
    out = _run(table, idx)
    return out.reshape(NH, N_NODES, D_FEAT).transpose(1, 0, 2)[None, None]
